# baseline (speedup 1.0000x reference)
import functools

import jax
import jax.numpy as jnp
from jax import lax
from jax.experimental import pallas as pl
from jax.experimental.pallas import tpu as pltpu
from jax.experimental.pallas import tpu_sc as plsc

SEQ = 200
BATCH = 4096
DIM = 64

NC = 2
NS = 16
NW = NC * NS
BT = 128
NBT = BATCH // BT
NCHUNK = SEQ * NBT
CPW = NCHUNK // NW

_MESH = plsc.VectorSubcoreMesh(
    core_axis_name="c", subcore_axis_name="s", num_cores=NC, num_subcores=NS
)


@functools.partial(
    pl.kernel,
    out_type=jax.ShapeDtypeStruct((SEQ, DIM // 8, NBT, 8, BT), jnp.float32),
    mesh=_MESH,
    compiler_params=pltpu.CompilerParams(
        use_tc_tiling_on_sc=False, needs_layout_passes=False
    ),
    scratch_types=[
        pltpu.VMEM((CPW, BT), jnp.int32),
        pltpu.VMEM((BT, DIM), jnp.float32),
        pltpu.VMEM((BT, DIM), jnp.float32),
        pltpu.VMEM((8, 8, BT), jnp.float32),
        pltpu.VMEM((8, 8, BT), jnp.float32),
        pltpu.SemaphoreType.DMA,
        pltpu.SemaphoreType.DMA,
        pltpu.SemaphoreType.DMA,
        pltpu.SemaphoreType.DMA,
    ],
)
def _gather_kernel(table_hbm, idx_hbm, out_hbm,
                   idx_v, rows0, rows1, t0, t1, g0, g1, w0, w1):
    wid = lax.axis_index("s") * NC + lax.axis_index("c")
    base = wid * CPW

    pltpu.sync_copy(idx_hbm.at[pl.ds(base, CPW)], idx_v)

    bufs = ((rows0, t0, g0, w0), (rows1, t1, g1, w1))

    lane = lax.iota(jnp.int32, 16)
    bcv = [lane + (bg * 16) for bg in range(8)]

    def fire_gather(i, rows, sem):
        pltpu.async_copy(table_hbm.at[idx_v.at[i]], rows, sem)

    def out_block(i):
        c = base + i
        return out_hbm.at[c // NBT, :, c % NBT]

    def transpose(rows, tbuf):
        @pl.loop(0, DIM)
        def _d(d):
            dv = jnp.full((16,), d, jnp.int32)
            dt = d // 8
            dr = d % 8
            for bg in range(8):
                v = plsc.load_gather(rows, [bcv[bg], dv])
                tbuf[dt, dr, pl.ds(bg * 16, 16)] = v

    fire_gather(0, rows0, g0)
    fire_gather(1, rows1, g1)

    def stage(i, rows, tbuf, gsem, wsem, *, first, last):
        pltpu.make_async_copy(table_hbm.at[idx_v.at[i]], rows, gsem).wait()
        if not first:
            pltpu.make_async_copy(tbuf, out_block(i - 2), wsem).wait()
        transpose(rows, tbuf)
        pltpu.async_copy(tbuf, out_block(i), wsem)
        if not last:
            fire_gather(i + 2, rows, gsem)

    stage(0, rows0, t0, g0, w0, first=True, last=False)
    stage(1, rows1, t1, g1, w1, first=True, last=False)

    @pl.loop(2, CPW - 2, step=2)
    def _chunks(i):
        for p, (rows, tbuf, gsem, wsem) in enumerate(bufs):
            stage(i + p, rows, tbuf, gsem, wsem, first=False, last=False)

    stage(CPW - 2, rows0, t0, g0, w0, first=False, last=True)
    stage(CPW - 1, rows1, t1, g1, w1, first=False, last=True)

    pltpu.make_async_copy(t0, out_block(CPW - 2), w0).wait()
    pltpu.make_async_copy(t1, out_block(CPW - 1), w1).wait()


def kernel(input, table):
    idx = input.reshape(NCHUNK, BT)
    out5 = _gather_kernel(table, idx)
    return out5.transpose(0, 2, 4, 1, 3).reshape(SEQ, BATCH, DIM)


# speedup vs baseline: 1.1615x; 1.1615x over previous
import functools

import jax
import jax.numpy as jnp
from jax import lax
from jax.experimental import pallas as pl
from jax.experimental.pallas import tpu as pltpu
from jax.experimental.pallas import tpu_sc as plsc

SEQ = 200
BATCH = 4096
DIM = 64

NC = 2
NS = 16
NW = NC * NS
BT = 128
NBT = BATCH // BT
NCHUNK = SEQ * NBT
CPW = NCHUNK // NW

_MESH = plsc.VectorSubcoreMesh(
    core_axis_name="c", subcore_axis_name="s", num_cores=NC, num_subcores=NS
)


@functools.partial(
    pl.kernel,
    out_type=jax.ShapeDtypeStruct((SEQ, DIM // 8, NBT, 8, BT), jnp.float32),
    mesh=_MESH,
    compiler_params=pltpu.CompilerParams(
        use_tc_tiling_on_sc=False, needs_layout_passes=False
    ),
    scratch_types=[
        pltpu.VMEM((CPW, BT), jnp.int32),
        pltpu.VMEM((BT, DIM), jnp.float32),
        pltpu.VMEM((BT, DIM), jnp.float32),
        pltpu.VMEM((8, 8, BT), jnp.float32),
        pltpu.VMEM((8, 8, BT), jnp.float32),
        pltpu.SemaphoreType.DMA,
        pltpu.SemaphoreType.DMA,
        pltpu.SemaphoreType.DMA,
        pltpu.SemaphoreType.DMA,
    ],
)
def _gather_kernel(table_hbm, idx_hbm, out_hbm,
                   idx_v, rows0, rows1, t0, t1, g0, g1, w0, w1):
    wid = lax.axis_index("s") * NC + lax.axis_index("c")
    base = wid * CPW

    pltpu.sync_copy(idx_hbm.at[pl.ds(base, CPW)], idx_v)

    bufs = ((rows0, t0, g0, w0), (rows1, t1, g1, w1))

    lane = lax.iota(jnp.int32, 16)
    bcv = [lane + (bg * 16) for bg in range(8)]
    dvs = [jnp.full((16,), d, jnp.int32) for d in range(DIM)]

    def fire_gather(i, rows, sem):
        pltpu.async_copy(table_hbm.at[idx_v.at[i]], rows, sem)

    def out_block(i):
        c = base + i
        return out_hbm.at[c // NBT, :, c % NBT]

    def transpose(rows, tbuf):
        WAVE = 32
        pairs = [(d, bg) for d in range(DIM) for bg in range(8)]
        for w0_ in range(0, len(pairs), WAVE):
            wave = pairs[w0_:w0_ + WAVE]
            vs = [plsc.load_gather(rows, [bcv[bg], dvs[d]]) for d, bg in wave]
            for (d, bg), v in zip(wave, vs):
                tbuf[d // 8, d % 8, pl.ds(bg * 16, 16)] = v

    fire_gather(0, rows0, g0)
    fire_gather(1, rows1, g1)

    @pl.loop(0, CPW, step=2)
    def _chunks(i):
        for p, (rows, tbuf, gsem, wsem) in enumerate(bufs):
            g = i + p
            pltpu.make_async_copy(table_hbm.at[idx_v.at[g]], rows, gsem).wait()

            @pl.when(g >= 2)
            def _():
                pltpu.make_async_copy(tbuf, out_block(g - 2), wsem).wait()

            transpose(rows, tbuf)
            pltpu.async_copy(tbuf, out_block(g), wsem)

            @pl.when(g + 2 < CPW)
            def _():
                fire_gather(g + 2, rows, gsem)

    pltpu.make_async_copy(t0, out_block(CPW - 2), w0).wait()
    pltpu.make_async_copy(t1, out_block(CPW - 1), w1).wait()


def kernel(input, table):
    idx = input.reshape(NCHUNK, BT)
    out5 = _gather_kernel(table, idx)
    return out5.transpose(0, 2, 4, 1, 3).reshape(SEQ, BATCH, DIM)


# speedup vs baseline: 1.5670x; 1.3492x over previous
import functools

import jax
import jax.numpy as jnp
from jax import lax
from jax.experimental import pallas as pl
from jax.experimental.pallas import tpu as pltpu
from jax.experimental.pallas import tpu_sc as plsc

SEQ = 200
BATCH = 4096
DIM = 64
VOCAB = 1000000

NC = 2
NS = 16
NW = NC * NS
BT = 128
NBT = BATCH // BT
NCHUNK = SEQ * NBT
CPW = NCHUNK // NW
TP = 129

_MESH = plsc.VectorSubcoreMesh(
    core_axis_name="c", subcore_axis_name="s", num_cores=NC, num_subcores=NS
)


@functools.partial(
    pl.kernel,
    out_type=jax.ShapeDtypeStruct((SEQ, DIM // 8, NBT, 8, BT), jnp.float32),
    mesh=_MESH,
    compiler_params=pltpu.CompilerParams(
        use_tc_tiling_on_sc=False, needs_layout_passes=False
    ),
    scratch_types=[
        pltpu.VMEM((CPW, BT), jnp.int32),
        pltpu.VMEM((CPW, BT), jnp.int32),
        pltpu.VMEM((BT, 128), jnp.float32),
        pltpu.VMEM((BT, 128), jnp.float32),
        pltpu.VMEM((8, 8, TP), jnp.float32),
        pltpu.VMEM((8, 8, TP), jnp.float32),
        pltpu.SemaphoreType.DMA,
        pltpu.SemaphoreType.DMA,
        pltpu.SemaphoreType.DMA,
        pltpu.SemaphoreType.DMA,
    ],
)
def _gather_kernel(table_hbm, idx_hbm, out_hbm,
                   idx_v, idx2_v, rows0, rows1, t0, t1, g0, g1, w0, w1):
    wid = lax.axis_index("s") * NC + lax.axis_index("c")
    base = wid * CPW

    pltpu.sync_copy(idx_hbm.at[pl.ds(base, CPW)], idx_v)

    @pl.loop(0, CPW)
    def _shift(i):
        for k in range(BT // 16):
            idx2_v[i, pl.ds(k * 16, 16)] = (
                lax.shift_right_logical(idx_v[i, pl.ds(k * 16, 16)], 1)
            )

    bufs = ((rows0, t0, g0, w0), (rows1, t1, g1, w1))

    lane = lax.iota(jnp.int32, 16)
    dtv = [(lane + 16 * gg) // 8 for gg in range(4)]
    drv = [(lane + 16 * gg) % 8 for gg in range(4)]
    bccv = [jnp.full((16,), bc, jnp.int32) for bc in range(BT)]

    def fire_gather(i, rows, sem):
        pltpu.async_copy(table_hbm.at[idx2_v.at[i]], rows, sem)

    def out_block(i):
        c = base + i
        return out_hbm.at[c // NBT, :, c % NBT]

    def transpose(g, rows, tbuf):
        for bg in range(8):
            pvv = (idx_v[g, pl.ds(bg * 16, 16)] & 1) * 64
            for half in range(2):
                wave = []
                for l in range(8 * half, 8 * half + 8):
                    bc = bg * 16 + l
                    off = pvv[l]
                    for gg in range(4):
                        wave.append(
                            (gg, bc, rows[bc, pl.ds(off + 16 * gg, 16)])
                        )
                for gg, bc, v in wave:
                    plsc.store_scatter(tbuf, [dtv[gg], drv[gg], bccv[bc]], v)

    fire_gather(0, rows0, g0)
    fire_gather(1, rows1, g1)

    @pl.loop(0, CPW, step=2)
    def _chunks(i):
        for p, (rows, tbuf, gsem, wsem) in enumerate(bufs):
            g = i + p
            pltpu.make_async_copy(table_hbm.at[idx2_v.at[g]], rows, gsem).wait()

            @pl.when(g >= 2)
            def _():
                pltpu.make_async_copy(
                    tbuf.at[:, :, pl.ds(0, BT)], out_block(g - 2), wsem
                ).wait()

            transpose(g, rows, tbuf)
            pltpu.async_copy(tbuf.at[:, :, pl.ds(0, BT)], out_block(g), wsem)

            @pl.when(g + 2 < CPW)
            def _():
                fire_gather(g + 2, rows, gsem)

    pltpu.make_async_copy(t0.at[:, :, pl.ds(0, BT)], out_block(CPW - 2), w0).wait()
    pltpu.make_async_copy(t1.at[:, :, pl.ds(0, BT)], out_block(CPW - 1), w1).wait()


def kernel(input, table):
    idx = input.reshape(NCHUNK, BT)
    table2 = table.reshape(VOCAB // 2, 128)
    out5 = _gather_kernel(table2, idx)
    return out5.transpose(0, 2, 4, 1, 3).reshape(SEQ, BATCH, DIM)


# speedup vs baseline: 2.3199x; 1.4805x over previous
import functools

import jax
import jax.numpy as jnp
from jax import lax
from jax.experimental import pallas as pl
from jax.experimental.pallas import tpu as pltpu
from jax.experimental.pallas import tpu_sc as plsc

SEQ = 200
BATCH = 4096
DIM = 64

NC = 2
NS = 16
NW = NC * NS
BT = 128
NBT = BATCH // BT
NCHUNK = SEQ * NBT
CPW = NCHUNK // NW
TP = 129

_MESH = plsc.VectorSubcoreMesh(
    core_axis_name="c", subcore_axis_name="s", num_cores=NC, num_subcores=NS
)


@functools.partial(
    pl.kernel,
    out_type=jax.ShapeDtypeStruct((SEQ, DIM // 8, NBT, 8, BT), jnp.float32),
    mesh=_MESH,
    compiler_params=pltpu.CompilerParams(
        use_tc_tiling_on_sc=False, needs_layout_passes=False
    ),
    scratch_types=[
        pltpu.VMEM((CPW, BT), jnp.int32),
        pltpu.VMEM((BT, DIM), jnp.float32),
        pltpu.VMEM((BT, DIM), jnp.float32),
        pltpu.VMEM((8, 8, TP), jnp.float32),
        pltpu.VMEM((8, 8, TP), jnp.float32),
        pltpu.SemaphoreType.DMA,
        pltpu.SemaphoreType.DMA,
        pltpu.SemaphoreType.DMA,
        pltpu.SemaphoreType.DMA,
    ],
)
def _gather_kernel(table_hbm, idx_hbm, out_hbm,
                   idx_v, rows0, rows1, t0, t1, g0, g1, w0, w1):
    wid = lax.axis_index("s") * NC + lax.axis_index("c")
    base = wid * CPW

    pltpu.sync_copy(idx_hbm.at[pl.ds(base, CPW)], idx_v)

    bufs = ((rows0, t0, g0, w0), (rows1, t1, g1, w1))

    lane = lax.iota(jnp.int32, 16)
    dtv = [(lane + 16 * gg) // 8 for gg in range(4)]
    drv = [(lane + 16 * gg) % 8 for gg in range(4)]
    bccv = [jnp.full((16,), bc, jnp.int32) for bc in range(BT)]

    def fire_gather(i, rows, sem):
        pltpu.async_copy(table_hbm.at[idx_v.at[i]], rows, sem)

    def out_block(i):
        c = base + i
        return out_hbm.at[c // NBT, :, c % NBT]

    def transpose(rows, tbuf):
        @plsc.parallel_loop(0, BT, unroll=4)
        def _bc(bc):
            bcv = lane * 0 + bc
            for gg in range(4):
                v = rows[bc, pl.ds(16 * gg, 16)]
                plsc.store_scatter(tbuf, [dtv[gg], drv[gg], bcv], v)

    fire_gather(0, rows0, g0)
    fire_gather(1, rows1, g1)

    @pl.loop(0, CPW, step=2)
    def _chunks(i):
        for p, (rows, tbuf, gsem, wsem) in enumerate(bufs):
            g = i + p
            pltpu.make_async_copy(table_hbm.at[idx_v.at[g]], rows, gsem).wait()

            @pl.when(g >= 2)
            def _():
                pltpu.make_async_copy(
                    tbuf.at[:, :, pl.ds(0, BT)], out_block(g - 2), wsem
                ).wait()

            transpose(rows, tbuf)
            pltpu.async_copy(tbuf.at[:, :, pl.ds(0, BT)], out_block(g), wsem)

            @pl.when(g + 2 < CPW)
            def _():
                fire_gather(g + 2, rows, gsem)

    pltpu.make_async_copy(t0.at[:, :, pl.ds(0, BT)], out_block(CPW - 2), w0).wait()
    pltpu.make_async_copy(t1.at[:, :, pl.ds(0, BT)], out_block(CPW - 1), w1).wait()


def kernel(input, table):
    idx = input.reshape(NCHUNK, BT)
    out5 = _gather_kernel(table, idx)
    return out5.transpose(0, 2, 4, 1, 3).reshape(SEQ, BATCH, DIM)


# speedup vs baseline: 2.4288x; 1.0469x over previous
import functools

import jax
import jax.numpy as jnp
from jax import lax
from jax.experimental import pallas as pl
from jax.experimental.pallas import tpu as pltpu
from jax.experimental.pallas import tpu_sc as plsc

SEQ = 200
BATCH = 4096
DIM = 64

NC = 2
NS = 16
NW = NC * NS
BT = 128
NBT = BATCH // BT
NCHUNK = SEQ * NBT
CPW = NCHUNK // NW
TP = 129

_MESH = plsc.VectorSubcoreMesh(
    core_axis_name="c", subcore_axis_name="s", num_cores=NC, num_subcores=NS
)


@functools.partial(
    pl.kernel,
    out_type=jax.ShapeDtypeStruct((SEQ, DIM // 8, NBT, 8, BT), jnp.float32),
    mesh=_MESH,
    compiler_params=pltpu.CompilerParams(
        use_tc_tiling_on_sc=False, needs_layout_passes=False
    ),
    scratch_types=[
        pltpu.VMEM((CPW, BT), jnp.int32),
    ] + [pltpu.VMEM((BT, DIM), jnp.float32) for _ in range(4)]
      + [pltpu.VMEM((8, 8, TP), jnp.float32) for _ in range(4)]
      + [pltpu.SemaphoreType.DMA for _ in range(8)],
)
def _gather_kernel(table_hbm, idx_hbm, out_hbm, idx_v,
                   rows0, rows1, rows2, rows3, t0, t1, t2, t3,
                   g0, g1, g2, g3, w0, w1, w2, w3):
    wid = lax.axis_index("s") * NC + lax.axis_index("c")
    base = wid * CPW

    pltpu.sync_copy(idx_hbm.at[pl.ds(base, CPW)], idx_v)

    bufs = ((rows0, t0, g0, w0), (rows1, t1, g1, w1),
            (rows2, t2, g2, w2), (rows3, t3, g3, w3))
    NBUF = len(bufs)

    lane = lax.iota(jnp.int32, 16)
    dtv = [(lane + 16 * gg) // 8 for gg in range(4)]
    drv = [(lane + 16 * gg) % 8 for gg in range(4)]
    bccv = [jnp.full((16,), bc, jnp.int32) for bc in range(BT)]

    def fire_gather(i, rows, sem):
        pltpu.async_copy(table_hbm.at[idx_v.at[i]], rows, sem)

    def out_block(i):
        c = base + i
        return out_hbm.at[c // NBT, :, c % NBT]

    def transpose(rows, tbuf):
        @plsc.parallel_loop(0, BT, unroll=4)
        def _bc(bc):
            bcv = lane * 0 + bc
            for gg in range(4):
                v = rows[bc, pl.ds(16 * gg, 16)]
                plsc.store_scatter(tbuf, [dtv[gg], drv[gg], bcv], v)

    for p, (rows, _, gsem, _) in enumerate(bufs):
        fire_gather(p, rows, gsem)

    @pl.loop(0, CPW, step=4)
    def _chunks(i):
        for p, (rows, tbuf, gsem, wsem) in enumerate(bufs):
            g = i + p
            pltpu.make_async_copy(table_hbm.at[idx_v.at[g]], rows, gsem).wait()

            @pl.when(g >= NBUF)
            def _():
                pltpu.make_async_copy(
                    tbuf.at[:, :, pl.ds(0, BT)], out_block(g - NBUF), wsem
                ).wait()

            transpose(rows, tbuf)
            pltpu.async_copy(tbuf.at[:, :, pl.ds(0, BT)], out_block(g), wsem)

            @pl.when(g + NBUF < CPW)
            def _():
                fire_gather(g + NBUF, rows, gsem)

    for p, (_, tbuf, _, wsem) in enumerate(bufs):
        pltpu.make_async_copy(
            tbuf.at[:, :, pl.ds(0, BT)], out_block(CPW - NBUF + p), wsem
        ).wait()


def kernel(input, table):
    idx = input.reshape(NCHUNK, BT)
    out5 = _gather_kernel(table, idx)
    return out5.transpose(0, 2, 4, 1, 3).reshape(SEQ, BATCH, DIM)


# speedup vs baseline: 2.4322x; 1.0014x over previous
import functools

import jax
import jax.numpy as jnp
from jax import lax
from jax.experimental import pallas as pl
from jax.experimental.pallas import tpu as pltpu
from jax.experimental.pallas import tpu_sc as plsc

SEQ = 200
BATCH = 4096
DIM = 64

NC = 2
NS = 16
NW = NC * NS
BT = 128
NBT = BATCH // BT
NCHUNK = SEQ * NBT
CPW = NCHUNK // NW
TP = 129

_MESH = plsc.VectorSubcoreMesh(
    core_axis_name="c", subcore_axis_name="s", num_cores=NC, num_subcores=NS
)


@functools.partial(
    pl.kernel,
    out_type=jax.ShapeDtypeStruct((SEQ, DIM // 8, NBT, 8, BT), jnp.float32),
    mesh=_MESH,
    compiler_params=pltpu.CompilerParams(
        use_tc_tiling_on_sc=False, needs_layout_passes=False
    ),
    scratch_types=[
        pltpu.VMEM((CPW, BT), jnp.int32),
    ] + [pltpu.VMEM((BT, DIM), jnp.float32) for _ in range(5)]
      + [pltpu.VMEM((8, 8, TP), jnp.float32) for _ in range(5)]
      + [pltpu.SemaphoreType.DMA for _ in range(10)],
)
def _gather_kernel(table_hbm, idx_hbm, out_hbm, idx_v,
                   rows0, rows1, rows2, rows3, rows4, t0, t1, t2, t3, t4,
                   g0, g1, g2, g3, g4, w0, w1, w2, w3, w4):
    wid = lax.axis_index("s") * NC + lax.axis_index("c")
    base = wid * CPW

    pltpu.sync_copy(idx_hbm.at[pl.ds(base, CPW)], idx_v)

    bufs = ((rows0, t0, g0, w0), (rows1, t1, g1, w1),
            (rows2, t2, g2, w2), (rows3, t3, g3, w3),
            (rows4, t4, g4, w4))
    NBUF = len(bufs)

    lane = lax.iota(jnp.int32, 16)
    dtv = [(lane + 16 * gg) // 8 for gg in range(4)]
    drv = [(lane + 16 * gg) % 8 for gg in range(4)]
    bccv = [jnp.full((16,), bc, jnp.int32) for bc in range(BT)]

    def fire_gather(i, rows, sem):
        pltpu.async_copy(table_hbm.at[idx_v.at[i]], rows, sem)

    def out_block(i):
        c = base + i
        return out_hbm.at[c // NBT, :, c % NBT]

    def transpose(rows, tbuf):
        @plsc.parallel_loop(0, BT, unroll=4)
        def _bc(bc):
            bcv = lane * 0 + bc
            for gg in range(4):
                v = rows[bc, pl.ds(16 * gg, 16)]
                plsc.store_scatter(tbuf, [dtv[gg], drv[gg], bcv], v)

    for p, (rows, _, gsem, _) in enumerate(bufs):
        fire_gather(p, rows, gsem)

    @pl.loop(0, CPW, step=5)
    def _chunks(i):
        for p, (rows, tbuf, gsem, wsem) in enumerate(bufs):
            g = i + p
            pltpu.make_async_copy(table_hbm.at[idx_v.at[g]], rows, gsem).wait()

            @pl.when(g >= NBUF)
            def _():
                pltpu.make_async_copy(
                    tbuf.at[:, :, pl.ds(0, BT)], out_block(g - NBUF), wsem
                ).wait()

            transpose(rows, tbuf)
            pltpu.async_copy(tbuf.at[:, :, pl.ds(0, BT)], out_block(g), wsem)

            @pl.when(g + NBUF < CPW)
            def _():
                fire_gather(g + NBUF, rows, gsem)

    for p, (_, tbuf, _, wsem) in enumerate(bufs):
        pltpu.make_async_copy(
            tbuf.at[:, :, pl.ds(0, BT)], out_block(CPW - NBUF + p), wsem
        ).wait()


def kernel(input, table):
    idx = input.reshape(NCHUNK, BT)
    out5 = _gather_kernel(table, idx)
    return out5.transpose(0, 2, 4, 1, 3).reshape(SEQ, BATCH, DIM)
